# baseline (speedup 1.0000x reference)
import jax
import jax.numpy as jnp
from jax import lax
from jax.experimental import pallas as pl
from jax.experimental.pallas import tpu as pltpu
from jax.experimental.pallas import tpu_sc as plsc

E = 320000
D = 128
N_NODES = 10000
NC = 2
NS = 16
NW = NC * NS
EPW = E // NW
B = 80
NB = EPW // B
NSLOT = 3
LANES = 16
ROWS_PER_TILE = 624


def _body(x_hbm, src_hbm, dst_hbm, out_hbm,
          si, di0, di1, di2, a0, a1, a2, xs,
          sg0, sg1, sg2, ss0, ss1, ss2, sd0, sd1, sd2):
    sid = lax.axis_index("s")
    wid = sid * NC + lax.axis_index("c")
    base = wid * EPW

    r0 = sid * ROWS_PER_TILE

    @pl.when(sid < NS - 1)
    def _():
        pltpu.sync_copy(x_hbm.at[pl.ds(r0, ROWS_PER_TILE)],
                        xs.at[pl.ds(r0, ROWS_PER_TILE)])

    @pl.when(sid == NS - 1)
    def _():
        t0 = (NS - 1) * ROWS_PER_TILE
        pltpu.sync_copy(x_hbm.at[pl.ds(t0, N_NODES - t0)],
                        xs.at[pl.ds(t0, N_NODES - t0)])

    pltpu.sync_copy(src_hbm.at[wid], si)
    plsc.subcore_barrier()

    a = (a0, a1, a2)
    di = (di0, di1, di2)
    sg = (sg0, sg1, sg2)
    ss = (ss0, ss1, ss2)
    sd = (sd0, sd1, sd2)

    def i_start(g, b):
        pltpu.async_copy(dst_hbm.at[wid * NB + g], di[b], sd[b])

    def i_wait(g, b):
        pltpu.make_async_copy(dst_hbm.at[wid * NB + g], di[b], sd[b]).wait()

    def g3_start(g, b):
        pltpu.async_copy(xs.at[di[b].at[0]], a[b], sg[b])

    def g3_wait(g, b):
        pltpu.make_async_copy(xs.at[di[b].at[0]], a[b], sg[b]).wait()

    def g1add_start(g, b):
        tbl = xs if b == 0 else x_hbm
        pltpu.async_copy(tbl.at[si.at[g]], a[b], sg[b], add=True)

    def g1add_wait(g, b):
        tbl = xs if b == 0 else x_hbm
        pltpu.make_async_copy(tbl.at[si.at[g]], a[b], sg[b]).wait()

    def s_start(g, b):
        pltpu.async_copy(a[b], out_hbm.at[pl.ds(base + g * B, B)], ss[b])

    def s_wait(g, b):
        pltpu.make_async_copy(a[b], out_hbm.at[pl.ds(base + g * B, B)],
                              ss[b]).wait()

    def negate(b):
        ab = a[b]

        def row(r, carry):
            for c in range(D // LANES):
                sl = pl.ds(c * LANES, LANES)
                ab[r, sl] = -ab[r, sl]
            return carry

        lax.fori_loop(0, B, row, 0, unroll=2)

    def steady(g, b):
        bp = (b - 1) % NSLOT
        bn = (b + 1) % NSLOT
        g3_wait(g, b)
        g1add_wait(g - 1, bp)
        s_start(g - 1, bp)
        i_wait(g + 1, bn)
        s_wait(g - 2, bn)
        g3_start(g + 1, bn)
        i_start(g + 2, bp)
        negate(b)
        g1add_start(g, b)

    i_start(0, 0)
    i_start(1, 1)
    i_wait(0, 0)
    g3_start(0, 0)

    g3_wait(0, 0)
    i_wait(1, 1)
    g3_start(1, 1)
    i_start(2, 2)
    negate(0)
    g1add_start(0, 0)

    g3_wait(1, 1)
    g1add_wait(0, 0)
    s_start(0, 0)
    i_wait(2, 2)
    g3_start(2, 2)
    i_start(3, 0)
    negate(1)
    g1add_start(1, 1)

    def tri(o, carry):
        for j in range(NSLOT):
            steady(3 * o + 2 + j, (2 + j) % NSLOT)
        return carry

    lax.fori_loop(0, (NB - 4) // NSLOT, tri, 0)
    for g in range(2 + 3 * ((NB - 4) // NSLOT), NB - 2):
        steady(g, g % NSLOT)

    g = NB - 2
    b = g % NSLOT
    g3_wait(g, b)
    g1add_wait(g - 1, (b - 1) % NSLOT)
    s_start(g - 1, (b - 1) % NSLOT)
    i_wait(g + 1, (b + 1) % NSLOT)
    s_wait(g - 2, (b + 1) % NSLOT)
    g3_start(g + 1, (b + 1) % NSLOT)
    negate(b)
    g1add_start(g, b)

    g = NB - 1
    b = g % NSLOT
    g3_wait(g, b)
    g1add_wait(g - 1, (b - 1) % NSLOT)
    s_start(g - 1, (b - 1) % NSLOT)
    s_wait(g - 2, (b + 1) % NSLOT)
    negate(b)
    g1add_start(g, b)

    g1add_wait(NB - 1, b)
    s_start(NB - 1, b)
    s_wait(NB - 2, (b - 1) % NSLOT)
    s_wait(NB - 1, b)


@jax.jit
def _sc_gather_sub(x, src3, dst3):
    mesh = plsc.VectorSubcoreMesh(core_axis_name="c", subcore_axis_name="s")
    return pl.kernel(
        _body,
        out_type=jax.ShapeDtypeStruct((E, D), jnp.float32),
        mesh=mesh,
        scratch_types=[
            pltpu.VMEM((NB, B), jnp.int32),
            pltpu.VMEM((1, B), jnp.int32),
            pltpu.VMEM((1, B), jnp.int32),
            pltpu.VMEM((1, B), jnp.int32),
            pltpu.VMEM((B, D), jnp.float32),
            pltpu.VMEM((B, D), jnp.float32),
            pltpu.VMEM((B, D), jnp.float32),
            pltpu.VMEM_SHARED((N_NODES, D), jnp.float32),
        ] + [pltpu.SemaphoreType.DMA] * 9,
    )(x, src3, dst3)


def kernel(x, edge_index):
    ei = edge_index.astype(jnp.int32)
    src3 = ei[0].reshape(NW, NB, B)
    dst3 = ei[1].reshape(NW * NB, 1, B)
    return _sc_gather_sub(x, src3, dst3)


# speedup vs baseline: 1.1882x; 1.1882x over previous
import jax
import jax.numpy as jnp
from jax import lax
from jax.experimental import pallas as pl
from jax.experimental.pallas import tpu as pltpu
from jax.experimental.pallas import tpu_sc as plsc

E = 320000
D = 128
N_NODES = 10000
NC = 2
NS = 16
NW = NC * NS
EPW = E // NW
B = 80
NB = EPW // B
NSLOT = 4
NDI = 4
ROWS_PER_TILE = 624


def _neg_body(x_ref, o_ref):
    o_ref[...] = -x_ref[...]


def _negate(x):
    return pl.pallas_call(
        _neg_body,
        out_shape=jax.ShapeDtypeStruct(x.shape, x.dtype),
    )(x)


def _body(x_hbm, xn_hbm, idx_hbm, out_hbm,
          di0, di1, di2, di3, a0, a1, a2, a3, xs,
          sg0, sg1, sg2, sg3, ss0, ss1, ss2, ss3, sd0, sd1, sd2, sd3):
    sid = lax.axis_index("s")
    wid = sid * NC + lax.axis_index("c")
    base = wid * EPW

    r0 = sid * ROWS_PER_TILE

    @pl.when(sid < NS - 1)
    def _():
        pltpu.sync_copy(x_hbm.at[pl.ds(r0, ROWS_PER_TILE)],
                        xs.at[pl.ds(r0, ROWS_PER_TILE)])

    @pl.when(sid == NS - 1)
    def _():
        t0 = (NS - 1) * ROWS_PER_TILE
        pltpu.sync_copy(x_hbm.at[pl.ds(t0, N_NODES - t0)],
                        xs.at[pl.ds(t0, N_NODES - t0)])

    plsc.subcore_barrier()

    a = (a0, a1, a2, a3)
    di = (di0, di1, di2, di3)
    sg = (sg0, sg1, sg2, sg3)
    ss = (ss0, ss1, ss2, ss3)
    sd = (sd0, sd1, sd2, sd3)

    def i_start(g, d):
        pltpu.async_copy(idx_hbm.at[wid * NB + g], di[d], sd[d])

    def i_wait(g, d):
        pltpu.make_async_copy(idx_hbm.at[wid * NB + g], di[d], sd[d]).wait()

    def g1_start(g, b):
        pltpu.async_copy(xs.at[di[b].at[0]], a[b], sg[b])

    def g1_wait(g, b):
        pltpu.make_async_copy(xs.at[di[b].at[0]], a[b], sg[b]).wait()

    def g2_start(g, b):
        pltpu.async_copy(xn_hbm.at[di[b].at[1]], a[b], sg[b], add=True)

    def g2_wait(g, b):
        pltpu.make_async_copy(xn_hbm.at[di[b].at[1]], a[b],
                              sg[b]).wait()

    def s_start(g, b):
        pltpu.async_copy(a[b], out_hbm.at[pl.ds(base + g * B, B)], ss[b])

    def s_wait(g, b):
        pltpu.make_async_copy(a[b], out_hbm.at[pl.ds(base + g * B, B)],
                              ss[b]).wait()

    def steady(g, b):
        i_wait(g + 1, (b + 1) % NSLOT)
        s_wait(g - 3, (b + 1) % NSLOT)
        g1_start(g + 1, (b + 1) % NSLOT)
        g1_wait(g, b)
        g2_start(g, b)
        g2_wait(g - 1, (b - 1) % NSLOT)
        s_start(g - 1, (b - 1) % NSLOT)
        i_start(g + 2, (b + 2) % NSLOT)

    i_start(0, 0)
    i_start(1, 1)
    i_wait(0, 0)
    g1_start(0, 0)
    i_start(2, 2)

    g1_wait(0, 0)
    g2_start(0, 0)
    i_wait(1, 1)
    g1_start(1, 1)
    i_start(3, 3)

    g1_wait(1, 1)
    g2_start(1, 1)
    g2_wait(0, 0)
    s_start(0, 0)
    i_wait(2, 2)
    g1_start(2, 2)
    i_start(4, 0)

    g1_wait(2, 2)
    g2_start(2, 2)
    g2_wait(1, 1)
    s_start(1, 1)
    i_wait(3, 3)
    g1_start(3, 3)

    def quad(o, carry):
        for j in range(NSLOT):
            steady(4 * o + 3 + j, (3 + j) % NSLOT)
        return carry

    lax.fori_loop(0, (NB - 5) // NSLOT, quad, 0)
    for g in range(3 + 4 * ((NB - 5) // NSLOT), NB - 2):
        steady(g, g % NSLOT)

    g = NB - 2
    b = g % NSLOT
    i_wait(g + 1, (b + 1) % NSLOT)
    s_wait(g - 3, (b + 1) % NSLOT)
    g1_start(g + 1, (b + 1) % NSLOT)
    g1_wait(g, b)
    g2_start(g, b)
    g2_wait(g - 1, (b - 1) % NSLOT)
    s_start(g - 1, (b - 1) % NSLOT)

    g = NB - 1
    b = g % NSLOT
    s_wait(g - 3, (b + 1) % NSLOT)
    g1_wait(g, b)
    g2_start(g, b)
    g2_wait(g - 1, (b - 1) % NSLOT)
    s_start(g - 1, (b - 1) % NSLOT)

    g2_wait(NB - 1, b)
    s_start(NB - 1, b)
    s_wait(NB - 3, (b - 2) % NSLOT)
    s_wait(NB - 2, (b - 1) % NSLOT)
    s_wait(NB - 1, b)


@jax.jit
def _sc_gather_sub(x, xneg, idx2):
    mesh = plsc.VectorSubcoreMesh(core_axis_name="c", subcore_axis_name="s")
    return pl.kernel(
        _body,
        out_type=jax.ShapeDtypeStruct((E, D), jnp.float32),
        mesh=mesh,
        scratch_types=[
            pltpu.VMEM((2, B), jnp.int32),
            pltpu.VMEM((2, B), jnp.int32),
            pltpu.VMEM((2, B), jnp.int32),
            pltpu.VMEM((2, B), jnp.int32),
            pltpu.VMEM((B, D), jnp.float32),
            pltpu.VMEM((B, D), jnp.float32),
            pltpu.VMEM((B, D), jnp.float32),
            pltpu.VMEM((B, D), jnp.float32),
            pltpu.VMEM_SHARED((N_NODES, D), jnp.float32),
        ] + [pltpu.SemaphoreType.DMA] * 12,
    )(x, xneg, idx2)


def kernel(x, edge_index):
    ei = edge_index.astype(jnp.int32)
    idx2 = jnp.stack(
        [ei[0].reshape(NW * NB, B), ei[1].reshape(NW * NB, B)], axis=1)
    return _sc_gather_sub(x, _negate(x), idx2)


# speedup vs baseline: 1.2932x; 1.0883x over previous
import jax
import jax.numpy as jnp
from jax import lax
from jax.experimental import pallas as pl
from jax.experimental.pallas import tpu as pltpu
from jax.experimental.pallas import tpu_sc as plsc

E = 320000
D = 128
N_NODES = 10000
NC = 2
NS = 16
NW = NC * NS
EPW = E // NW
B = 80
NB = EPW // B
NSLOT = 3
ROWS_PER_TILE = 624


def _neg_body(x_ref, o_ref):
    o_ref[...] = -x_ref[...]


def _negate(x):
    return pl.pallas_call(
        _neg_body,
        out_shape=jax.ShapeDtypeStruct(x.shape, x.dtype),
    )(x)


def _body(x_hbm, xn_hbm, src_hbm, dst_hbm, out_hbm,
          si, di0, di1, di2, a0, a1, a2, xs,
          sg0, sg1, sg2, ss0, ss1, ss2, sd0, sd1, sd2):
    sid = lax.axis_index("s")
    wid = sid * NC + lax.axis_index("c")
    base = wid * EPW

    r0 = sid * ROWS_PER_TILE

    @pl.when(sid < NS - 1)
    def _():
        pltpu.sync_copy(x_hbm.at[pl.ds(r0, ROWS_PER_TILE)],
                        xs.at[pl.ds(r0, ROWS_PER_TILE)])

    @pl.when(sid == NS - 1)
    def _():
        t0 = (NS - 1) * ROWS_PER_TILE
        pltpu.sync_copy(x_hbm.at[pl.ds(t0, N_NODES - t0)],
                        xs.at[pl.ds(t0, N_NODES - t0)])

    pltpu.sync_copy(src_hbm.at[wid], si)
    plsc.subcore_barrier()

    a = (a0, a1, a2)
    di = (di0, di1, di2)
    sg = (sg0, sg1, sg2)
    ss = (ss0, ss1, ss2)
    sd = (sd0, sd1, sd2)

    def ig_start(g, b):
        pltpu.async_copy(dst_hbm.at[wid * NB + g], di[b], sd[b])
        pltpu.async_copy(xs.at[si.at[g]], a[b], sg[b])

    def ig_wait(g, b):
        pltpu.make_async_copy(dst_hbm.at[wid * NB + g], di[b], sd[b]).wait()
        pltpu.make_async_copy(xs.at[si.at[g]], a[b], sg[b]).wait()

    def g2_start(g, b):
        pltpu.async_copy(xn_hbm.at[di[b].at[0]], a[b], sg[b], add=True)

    def g2_wait(g, b):
        pltpu.make_async_copy(xn_hbm.at[di[b].at[0]], a[b], sg[b]).wait()

    def s_start(g, b):
        pltpu.async_copy(a[b], out_hbm.at[pl.ds(base + g * B, B)], ss[b])

    def s_wait(g, b):
        pltpu.make_async_copy(a[b], out_hbm.at[pl.ds(base + g * B, B)],
                              ss[b]).wait()

    def steady(g, b):
        ig_wait(g, b)
        g2_start(g, b)
        s_wait(g - 2, (b - 2) % NSLOT)
        ig_start(g + 1, (b + 1) % NSLOT)
        g2_wait(g - 1, (b - 1) % NSLOT)
        s_start(g - 1, (b - 1) % NSLOT)

    ig_start(0, 0)
    ig_wait(0, 0)
    g2_start(0, 0)
    ig_start(1, 1)

    ig_wait(1, 1)
    g2_start(1, 1)
    g2_wait(0, 0)
    s_start(0, 0)
    ig_start(2, 2)

    def tri(o, carry):
        for j in range(NSLOT):
            steady(3 * o + 2 + j, (2 + j) % NSLOT)
        return carry

    lax.fori_loop(0, (NB - 3) // NSLOT, tri, 0)
    for g in range(2 + 3 * ((NB - 3) // NSLOT), NB - 1):
        steady(g, g % NSLOT)

    bl = (NB - 1) % NSLOT
    ig_wait(NB - 1, bl)
    g2_start(NB - 1, bl)
    g2_wait(NB - 2, (bl - 1) % NSLOT)
    s_start(NB - 2, (bl - 1) % NSLOT)
    s_wait(NB - 3, (bl - 2) % NSLOT)

    g2_wait(NB - 1, bl)
    s_start(NB - 1, bl)
    s_wait(NB - 2, (bl - 1) % NSLOT)
    s_wait(NB - 1, bl)


@jax.jit
def _sc_gather_sub(x, xneg, src3, dst3):
    mesh = plsc.VectorSubcoreMesh(core_axis_name="c", subcore_axis_name="s")
    return pl.kernel(
        _body,
        out_type=jax.ShapeDtypeStruct((E, D), jnp.float32),
        mesh=mesh,
        scratch_types=[
            pltpu.VMEM((NB, B), jnp.int32),
            pltpu.VMEM((1, B), jnp.int32),
            pltpu.VMEM((1, B), jnp.int32),
            pltpu.VMEM((1, B), jnp.int32),
            pltpu.VMEM((B, D), jnp.float32),
            pltpu.VMEM((B, D), jnp.float32),
            pltpu.VMEM((B, D), jnp.float32),
            pltpu.VMEM_SHARED((N_NODES, D), jnp.float32),
        ] + [pltpu.SemaphoreType.DMA] * 9,
    )(x, xneg, src3, dst3)


def kernel(x, edge_index):
    ei = edge_index.astype(jnp.int32)
    src3 = ei[0].reshape(NW, NB, B)
    dst3 = ei[1].reshape(NW * NB, 1, B)
    return _sc_gather_sub(x, _negate(x), src3, dst3)
